# initial kernel scaffold (unmeasured)
import jax
import jax.numpy as jnp
from jax import lax
from jax.experimental import pallas as pl
from jax.experimental.pallas import tpu as pltpu

N = 4
T_LOC = 128
D = 512
E = 8
E_LOC = 2
F = 1024


def kernel(x, router, W1, W2):
    def body(x_ref, r_ref, w1_ref, w2_ref, out_ref,
             agbuf, rbuf, xall, rall, pstore, sbuf, rrbuf,
             ag_send, ag_recv, r_send, r_recv, rs_send, rs_recv):
        my = lax.axis_index("y")
        mx = lax.axis_index("x")
        mz = lax.axis_index("z")
        right = (my + 1) % N
        left = (my + N - 1) % N

        barrier = pltpu.get_barrier_semaphore()
        for nbr in (left, right):
            pl.semaphore_signal(
                barrier, inc=1,
                device_id=(mx, nbr, mz),
                device_id_type=pl.DeviceIdType.MESH,
            )
        pl.semaphore_wait(barrier, 2)

        agbuf[0] = x_ref[...]
        rbuf[0] = r_ref[...]
        xall[pl.ds(my * T_LOC, T_LOC), :] = x_ref[...]
        rall[pl.ds(my, 1)] = r_ref[...][None]

        for h in range(N - 1):
            rx = pltpu.make_async_remote_copy(
                src_ref=agbuf.at[h], dst_ref=agbuf.at[h + 1],
                send_sem=ag_send.at[h], recv_sem=ag_recv.at[h],
                device_id=(mx, right, mz),
                device_id_type=pl.DeviceIdType.MESH,
            )
            rr = pltpu.make_async_remote_copy(
                src_ref=rbuf.at[h], dst_ref=rbuf.at[h + 1],
                send_sem=r_send.at[h], recv_sem=r_recv.at[h],
                device_id=(mx, right, mz),
                device_id_type=pl.DeviceIdType.MESH,
            )
            rx.start()
            rr.start()
            rx.wait()
            rr.wait()
            o = (my - 1 - h) % N
            xall[pl.ds(o * T_LOC, T_LOC), :] = agbuf[h + 1]
            rall[pl.ds(o, 1)] = rbuf[h + 1][None]

        X = xall[...]
        R = jnp.concatenate([rall[j] for j in range(N)], axis=1)
        g = jax.lax.dot_general(
            X, R, (((1,), (0,)), ((), ())),
            preferred_element_type=jnp.float32,
        )
        top1 = jnp.max(g, axis=1, keepdims=True)
        is1 = g == top1
        g2 = jnp.where(is1, -jnp.inf, g)
        top2 = jnp.max(g2, axis=1, keepdims=True)
        b = jnp.exp(top2 - top1)
        w_top1 = 1.0 / (1.0 + b)
        w_top2 = b / (1.0 + b)
        W8 = jnp.where(is1, w_top1, jnp.where(g2 == top2, w_top2, 0.0))

        eids = lax.broadcasted_iota(jnp.int32, (T_LOC * N, E), 1)
        P = jnp.zeros((T_LOC * N, D), dtype=jnp.float32)
        for j in range(E_LOC):
            wj = jnp.sum(
                jnp.where(eids == E_LOC * my + j, W8, 0.0),
                axis=1, keepdims=True,
            )
            h1 = jax.lax.dot_general(
                X, w1_ref[j], (((1,), (0,)), ((), ())),
                preferred_element_type=jnp.float32,
            )
            h1 = jnp.maximum(h1, 0.0) * wj
            P = P + jax.lax.dot_general(
                h1, w2_ref[j], (((1,), (0,)), ((), ())),
                preferred_element_type=jnp.float32,
            )
        pstore[...] = P

        for s in range(N - 1):
            c = (my - 1 - s) % N
            chunk = pstore[pl.ds(c * T_LOC, T_LOC), :]
            if s == 0:
                sbuf[s] = chunk
            else:
                sbuf[s] = chunk + rrbuf[s - 1]
            rs = pltpu.make_async_remote_copy(
                src_ref=sbuf.at[s], dst_ref=rrbuf.at[s],
                send_sem=rs_send.at[s], recv_sem=rs_recv.at[s],
                device_id=(mx, right, mz),
                device_id_type=pl.DeviceIdType.MESH,
            )
            rs.start()
            rs.wait()

        out_ref[...] = pstore[pl.ds(my * T_LOC, T_LOC), :] + rrbuf[N - 2]

    return pl.pallas_call(
        body,
        out_shape=jax.ShapeDtypeStruct((T_LOC, D), jnp.float32),
        in_specs=[pl.BlockSpec(memory_space=pltpu.VMEM)] * 4,
        out_specs=pl.BlockSpec(memory_space=pltpu.VMEM),
        scratch_shapes=[
            pltpu.VMEM((N, T_LOC, D), jnp.float32),
            pltpu.VMEM((N, D, E_LOC), jnp.float32),
            pltpu.VMEM((N * T_LOC, D), jnp.float32),
            pltpu.VMEM((N, D, E_LOC), jnp.float32),
            pltpu.VMEM((N * T_LOC, D), jnp.float32),
            pltpu.VMEM((N - 1, T_LOC, D), jnp.float32),
            pltpu.VMEM((N - 1, T_LOC, D), jnp.float32),
            pltpu.SemaphoreType.DMA((N - 1,)),
            pltpu.SemaphoreType.DMA((N - 1,)),
            pltpu.SemaphoreType.DMA((N - 1,)),
            pltpu.SemaphoreType.DMA((N - 1,)),
            pltpu.SemaphoreType.DMA((N - 1,)),
            pltpu.SemaphoreType.DMA((N - 1,)),
        ],
        compiler_params=pltpu.CompilerParams(collective_id=0),
    )(x, router, W1, W2)


# baseline (device time: 63646 ns/iter reference)
import jax
import jax.numpy as jnp
from jax import lax
from jax.experimental import pallas as pl
from jax.experimental.pallas import tpu as pltpu

N = 4
T_LOC = 128
D = 512
E = 8
E_LOC = 2
F = 1024


def kernel(x, router, W1, W2):
    def body(x_ref, r_ref, w1_ref, w2_ref, out_ref,
             agbuf, rbuf, xall, rall, pstore, sbuf, rrbuf,
             ag_send, ag_recv, r_send, r_recv, rs_send, rs_recv):
        my = lax.axis_index("y")
        mx = lax.axis_index("x")
        mz = lax.axis_index("z")
        right = (my + 1) % N
        left = (my + N - 1) % N

        barrier = pltpu.get_barrier_semaphore()
        for nbr in (left, right):
            pl.semaphore_signal(
                barrier, inc=1,
                device_id=(mx, nbr, mz),
                device_id_type=pl.DeviceIdType.MESH,
            )
        pl.semaphore_wait(barrier, 2)

        agbuf[0] = x_ref[...]
        rbuf[0] = r_ref[...]
        xall[pl.ds(my * T_LOC, T_LOC), :] = x_ref[...]
        rall[pl.ds(my, 1)] = r_ref[...][None]

        for h in range(N - 1):
            rx = pltpu.make_async_remote_copy(
                src_ref=agbuf.at[h], dst_ref=agbuf.at[h + 1],
                send_sem=ag_send.at[h], recv_sem=ag_recv.at[h],
                device_id=(mx, right, mz),
                device_id_type=pl.DeviceIdType.MESH,
            )
            rr = pltpu.make_async_remote_copy(
                src_ref=rbuf.at[h], dst_ref=rbuf.at[h + 1],
                send_sem=r_send.at[h], recv_sem=r_recv.at[h],
                device_id=(mx, right, mz),
                device_id_type=pl.DeviceIdType.MESH,
            )
            rx.start()
            rr.start()
            rx.wait()
            rr.wait()
            o = (my - 1 - h) % N
            xall[pl.ds(o * T_LOC, T_LOC), :] = agbuf[h + 1]
            rall[pl.ds(o, 1)] = rbuf[h + 1][None]

        X = xall[...]
        R = jnp.concatenate([rall[j] for j in range(N)], axis=1)
        g = jax.lax.dot_general(
            X, R, (((1,), (0,)), ((), ())),
            preferred_element_type=jnp.float32,
            precision=jax.lax.Precision.HIGHEST,
        )
        top1 = jnp.max(g, axis=1, keepdims=True)
        is1 = g == top1
        g2 = jnp.where(is1, -jnp.inf, g)
        top2 = jnp.max(g2, axis=1, keepdims=True)
        b = jnp.exp(top2 - top1)
        w_top1 = 1.0 / (1.0 + b)
        w_top2 = b / (1.0 + b)
        W8 = jnp.where(is1, w_top1, jnp.where(g2 == top2, w_top2, 0.0))

        eids = lax.broadcasted_iota(jnp.int32, (T_LOC * N, E), 1)
        P = jnp.zeros((T_LOC * N, D), dtype=jnp.float32)
        for j in range(E_LOC):
            wj = jnp.sum(
                jnp.where(eids == E_LOC * my + j, W8, 0.0),
                axis=1, keepdims=True,
            )
            h1 = jax.lax.dot_general(
                X, w1_ref[j], (((1,), (0,)), ((), ())),
                preferred_element_type=jnp.float32,
                precision=jax.lax.Precision.HIGHEST,
            )
            h1 = jnp.maximum(h1, 0.0) * wj
            P = P + jax.lax.dot_general(
                h1, w2_ref[j], (((1,), (0,)), ((), ())),
                preferred_element_type=jnp.float32,
                precision=jax.lax.Precision.HIGHEST,
            )
        pstore[...] = P

        for s in range(N - 1):
            c = (my - 1 - s) % N
            chunk = pstore[pl.ds(c * T_LOC, T_LOC), :]
            if s == 0:
                sbuf[s] = chunk
            else:
                sbuf[s] = chunk + rrbuf[s - 1]
            rs = pltpu.make_async_remote_copy(
                src_ref=sbuf.at[s], dst_ref=rrbuf.at[s],
                send_sem=rs_send.at[s], recv_sem=rs_recv.at[s],
                device_id=(mx, right, mz),
                device_id_type=pl.DeviceIdType.MESH,
            )
            rs.start()
            rs.wait()

        out_ref[...] = pstore[pl.ds(my * T_LOC, T_LOC), :] + rrbuf[N - 2]

    return pl.pallas_call(
        body,
        out_shape=jax.ShapeDtypeStruct((T_LOC, D), jnp.float32),
        in_specs=[pl.BlockSpec(memory_space=pltpu.VMEM)] * 4,
        out_specs=pl.BlockSpec(memory_space=pltpu.VMEM),
        scratch_shapes=[
            pltpu.VMEM((N, T_LOC, D), jnp.float32),
            pltpu.VMEM((N, D, E_LOC), jnp.float32),
            pltpu.VMEM((N * T_LOC, D), jnp.float32),
            pltpu.VMEM((N, D, E_LOC), jnp.float32),
            pltpu.VMEM((N * T_LOC, D), jnp.float32),
            pltpu.VMEM((N - 1, T_LOC, D), jnp.float32),
            pltpu.VMEM((N - 1, T_LOC, D), jnp.float32),
            pltpu.SemaphoreType.DMA((N - 1,)),
            pltpu.SemaphoreType.DMA((N - 1,)),
            pltpu.SemaphoreType.DMA((N - 1,)),
            pltpu.SemaphoreType.DMA((N - 1,)),
            pltpu.SemaphoreType.DMA((N - 1,)),
            pltpu.SemaphoreType.DMA((N - 1,)),
        ],
        compiler_params=pltpu.CompilerParams(collective_id=0),
    )(x, router, W1, W2)
